# baseline (speedup 1.0000x reference)
import functools

import numpy as np
import jax
import jax.numpy as jnp
from jax import lax
from jax.experimental import pallas as pl
from jax.experimental.pallas import tpu as pltpu
from jax.experimental.pallas import tpu_sc as plsc

_SEED = 1337
_B, _N = 128, 32768
_NW = 32
_ROWS_PER_W = _B // _NW
_L = 16
_OC = 8192
_NCHUNK = _N // _OC


def _threefry2x32(k1, k2, x1, x2):
    rotations = ((13, 15, 26, 6), (17, 29, 16, 24))
    ks = (k1, k2, np.uint32(k1 ^ k2 ^ np.uint32(0x1BD11BDA)))
    x1 = (x1 + ks[0]).astype(np.uint32)
    x2 = (x2 + ks[1]).astype(np.uint32)
    for i in range(5):
        for r in rotations[i % 2]:
            x1 = (x1 + x2).astype(np.uint32)
            x2 = np.uint32((x2 << np.uint32(r)) | (x2 >> np.uint32(32 - r)))
            x2 = x1 ^ x2
        x1 = (x1 + ks[(i + 1) % 3]).astype(np.uint32)
        x2 = (x2 + ks[(i + 2) % 3] + np.uint32(i + 1)).astype(np.uint32)
    return x1, x2


def _inv_perm() -> np.ndarray:
    counts = np.arange(_N, dtype=np.uint64)
    hi = (counts >> np.uint64(32)).astype(np.uint32)
    lo = counts.astype(np.uint32)
    o1, o2 = _threefry2x32(
        np.uint32(_SEED >> 32), np.uint32(_SEED & 0xFFFFFFFF), hi, lo)
    bits = o1 ^ o2
    noise = ((bits >> np.uint32(9)) | np.uint32(0x3F800000)).view(
        np.float32) - np.float32(1.0)
    noise = np.maximum(np.float32(0.0), noise)
    perm = np.argsort(noise, kind="stable")
    return np.argsort(perm, kind="stable").astype(np.int32)


def _packed_idx() -> np.ndarray:
    idx2 = _inv_perm().astype(np.uint32).reshape(-1, 2, 16)
    packed = idx2[:, 0, :] | (idx2[:, 1, :] << np.uint32(16))
    return packed.reshape(-1).view(np.int32).copy()


_PACKED_IDX = _packed_idx()
_DEINT_CACHE = None


def _build_deinterleave():
    @functools.partial(
        pl.kernel,
        out_type=jax.ShapeDtypeStruct((_B, _N), jnp.float32),
        mesh=plsc.VectorSubcoreMesh(core_axis_name="c", subcore_axis_name="s"),
        compiler_params=pltpu.CompilerParams(needs_layout_passes=False),
        scratch_types=[
            pltpu.VMEM((_N // 2,), jnp.int32),
            pltpu.VMEM((_N,), jnp.float32),
            pltpu.VMEM((_N,), jnp.float32),
            pltpu.VMEM((_OC,), jnp.float32),
            pltpu.VMEM((_OC,), jnp.float32),
            pltpu.SemaphoreType.DMA,
            pltpu.SemaphoreType.DMA,
            pltpu.SemaphoreType.DMA,
            pltpu.SemaphoreType.DMA,
            pltpu.SemaphoreType.DMA,
            pltpu.SemaphoreType.DMA,
            pltpu.SemaphoreType.DMA,
            pltpu.SemaphoreType.DMA,
        ],
    )
    def _deinterleave(x_hbm, idx_hbm, out_hbm, idx_v, row0_v, row1_v,
                      out0_v, out1_v, idx_sem0, idx_sem1, idx_sem2, idx_sem3,
                      in_sem0, in_sem1, out_sem0, out_sem1):
        wid = lax.axis_index("s") * 2 + lax.axis_index("c")
        base_row = wid * _ROWS_PER_W
        row_bufs = (row0_v, row1_v)
        out_bufs = (out0_v, out1_v)
        in_sems = (in_sem0, in_sem1)
        out_sems = (out_sem0, out_sem1)
        idx_sems = (idx_sem0, idx_sem1, idx_sem2, idx_sem3)

        in_cps = [None, None]
        in_cps[0] = pltpu.async_copy(
            x_hbm.at[base_row], row_bufs[0], in_sems[0])
        _WC = _OC // 2
        idx_cps = [
            pltpu.async_copy(
                idx_hbm.at[pl.ds(c * _WC, _WC)],
                idx_v.at[pl.ds(c * _WC, _WC)],
                idx_sems[c])
            for c in range(_NCHUNK)
        ]
        out_cps = [None, None]

        for r in range(_ROWS_PER_W):
            rb = r % 2
            in_cps[rb].wait()
            if r + 1 < _ROWS_PER_W:
                nb = (r + 1) % 2
                in_cps[nb] = pltpu.async_copy(
                    x_hbm.at[base_row + r + 1], row_bufs[nb], in_sems[nb])
            row_ref = row_bufs[rb]
            for c in range(_NCHUNK):
                p = (r * _NCHUNK + c) % 2
                if out_cps[p] is not None:
                    out_cps[p].wait()
                if r == 0 and idx_cps[c] is not None:
                    idx_cps[c].wait()
                    idx_cps[c] = None
                out_ref = out_bufs[p]
                off = c * _OC
                woff = off // 2

                @plsc.parallel_loop(0, _OC, 2 * _L, unroll=8)
                def _gather_chunk(b):
                    w16 = idx_v[pl.ds(woff + lax.shift_right_logical(b, 1),
                                      _L)]
                    ia = lax.bitwise_and(w16, jnp.int32(0xFFFF))
                    ib = lax.shift_right_logical(w16, jnp.int32(16))
                    out_ref[pl.ds(b, _L)] = plsc.load_gather(row_ref, [ia])
                    out_ref[pl.ds(b + _L, _L)] = plsc.load_gather(
                        row_ref, [ib])
                out_cps[p] = pltpu.async_copy(
                    out_ref,
                    out_hbm.at[base_row + r, pl.ds(off, _OC)],
                    out_sems[p])
        for p in range(2):
            if out_cps[p] is not None:
                out_cps[p].wait()

    return _deinterleave


def kernel(x):
    global _DEINT_CACHE
    if _DEINT_CACHE is None:
        _DEINT_CACHE = _build_deinterleave()
    return _DEINT_CACHE(x, jnp.asarray(_PACKED_IDX))


# speedup vs baseline: 1.0289x; 1.0289x over previous
import functools

import numpy as np
import jax
import jax.numpy as jnp
from jax import lax
from jax.experimental import pallas as pl
from jax.experimental.pallas import tpu as pltpu
from jax.experimental.pallas import tpu_sc as plsc

_SEED = 1337
_B, _N = 128, 32768
_NW = 32
_ROWS_PER_W = _B // _NW
_L = 16
_OC = 16384
_NCHUNK = _N // _OC


def _threefry2x32(k1, k2, x1, x2):
    rotations = ((13, 15, 26, 6), (17, 29, 16, 24))
    ks = (k1, k2, np.uint32(k1 ^ k2 ^ np.uint32(0x1BD11BDA)))
    x1 = (x1 + ks[0]).astype(np.uint32)
    x2 = (x2 + ks[1]).astype(np.uint32)
    for i in range(5):
        for r in rotations[i % 2]:
            x1 = (x1 + x2).astype(np.uint32)
            x2 = np.uint32((x2 << np.uint32(r)) | (x2 >> np.uint32(32 - r)))
            x2 = x1 ^ x2
        x1 = (x1 + ks[(i + 1) % 3]).astype(np.uint32)
        x2 = (x2 + ks[(i + 2) % 3] + np.uint32(i + 1)).astype(np.uint32)
    return x1, x2


def _inv_perm() -> np.ndarray:
    counts = np.arange(_N, dtype=np.uint64)
    hi = (counts >> np.uint64(32)).astype(np.uint32)
    lo = counts.astype(np.uint32)
    o1, o2 = _threefry2x32(
        np.uint32(_SEED >> 32), np.uint32(_SEED & 0xFFFFFFFF), hi, lo)
    bits = o1 ^ o2
    noise = ((bits >> np.uint32(9)) | np.uint32(0x3F800000)).view(
        np.float32) - np.float32(1.0)
    noise = np.maximum(np.float32(0.0), noise)
    perm = np.argsort(noise, kind="stable")
    return np.argsort(perm, kind="stable").astype(np.int32)


def _packed_idx() -> np.ndarray:
    idx2 = _inv_perm().astype(np.uint32).reshape(-1, 2, 16)
    packed = idx2[:, 0, :] | (idx2[:, 1, :] << np.uint32(16))
    return packed.reshape(-1).view(np.int32).copy()


_PACKED_IDX = _packed_idx()
_DEINT_CACHE = None


def _build_deinterleave():
    @functools.partial(
        pl.kernel,
        out_type=jax.ShapeDtypeStruct((_B, _N), jnp.float32),
        mesh=plsc.VectorSubcoreMesh(core_axis_name="c", subcore_axis_name="s"),
        compiler_params=pltpu.CompilerParams(needs_layout_passes=False),
        scratch_types=[
            pltpu.VMEM((_N // 2,), jnp.int32),
            pltpu.VMEM((_N,), jnp.float32),
            pltpu.VMEM((_N,), jnp.float32),
            pltpu.VMEM((_OC,), jnp.float32),
            pltpu.VMEM((_OC,), jnp.float32),
            pltpu.SemaphoreType.DMA,
            pltpu.SemaphoreType.DMA,
            pltpu.SemaphoreType.DMA,
            pltpu.SemaphoreType.DMA,
            pltpu.SemaphoreType.DMA,
            pltpu.SemaphoreType.DMA,
            pltpu.SemaphoreType.DMA,
            pltpu.SemaphoreType.DMA,
        ],
    )
    def _deinterleave(x_hbm, idx_hbm, out_hbm, idx_v, row0_v, row1_v,
                      out0_v, out1_v, idx_sem0, idx_sem1, idx_sem2, idx_sem3,
                      in_sem0, in_sem1, out_sem0, out_sem1):
        wid = lax.axis_index("s") * 2 + lax.axis_index("c")
        base_row = wid * _ROWS_PER_W
        row_bufs = (row0_v, row1_v)
        out_bufs = (out0_v, out1_v)
        in_sems = (in_sem0, in_sem1)
        out_sems = (out_sem0, out_sem1)
        idx_sems = (idx_sem0, idx_sem1, idx_sem2, idx_sem3)

        in_cps = [None, None]
        in_cps[0] = pltpu.async_copy(
            x_hbm.at[base_row], row_bufs[0], in_sems[0])
        _WC = _OC // 2
        idx_cps = [
            pltpu.async_copy(
                idx_hbm.at[pl.ds(c * _WC, _WC)],
                idx_v.at[pl.ds(c * _WC, _WC)],
                idx_sems[c])
            for c in range(_NCHUNK)
        ]
        out_cps = [None, None]

        for r in range(_ROWS_PER_W):
            rb = r % 2
            in_cps[rb].wait()
            if r + 1 < _ROWS_PER_W:
                nb = (r + 1) % 2
                in_cps[nb] = pltpu.async_copy(
                    x_hbm.at[base_row + r + 1], row_bufs[nb], in_sems[nb])
            row_ref = row_bufs[rb]
            for c in range(_NCHUNK):
                p = (r * _NCHUNK + c) % 2
                if out_cps[p] is not None:
                    out_cps[p].wait()
                if r == 0 and idx_cps[c] is not None:
                    idx_cps[c].wait()
                    idx_cps[c] = None
                out_ref = out_bufs[p]
                off = c * _OC
                woff = off // 2

                @plsc.parallel_loop(0, _OC, 2 * _L, unroll=8)
                def _gather_chunk(b):
                    w16 = idx_v[pl.ds(woff + lax.shift_right_logical(b, 1),
                                      _L)]
                    ia = lax.bitwise_and(w16, jnp.int32(0xFFFF))
                    ib = lax.shift_right_logical(w16, jnp.int32(16))
                    out_ref[pl.ds(b, _L)] = plsc.load_gather(row_ref, [ia])
                    out_ref[pl.ds(b + _L, _L)] = plsc.load_gather(
                        row_ref, [ib])
                out_cps[p] = pltpu.async_copy(
                    out_ref,
                    out_hbm.at[base_row + r, pl.ds(off, _OC)],
                    out_sems[p])
        for p in range(2):
            if out_cps[p] is not None:
                out_cps[p].wait()

    return _deinterleave


def kernel(x):
    global _DEINT_CACHE
    if _DEINT_CACHE is None:
        _DEINT_CACHE = _build_deinterleave()
    return _DEINT_CACHE(x, jnp.asarray(_PACKED_IDX))
